# baseline (speedup 1.0000x reference)
import functools

import jax
import jax.numpy as jnp
from jax.experimental import pallas as pl
from jax.experimental.pallas import tpu as pltpu

_BW = 512


def _body(feats_ref, tgt_ref, queue_ref, qlab_ref, out_ref, lab_ref, *, nfb):
    j = pl.program_id(0)

    @pl.when(j < nfb)
    def _insert():
        out_ref[...] = feats_ref[...].T
        lab_ref[...] = tgt_ref[...]

    @pl.when(j >= nfb)
    def _copy():
        out_ref[...] = queue_ref[...]
        lab_ref[...] = qlab_ref[...]


def kernel(feats, targets, queue, queue_label):
    bsz, dim = feats.shape
    k = queue.shape[1]
    nfb = bsz // _BW
    nblocks = k // _BW
    targets2d = targets.reshape(1, bsz)

    body = functools.partial(_body, nfb=nfb)

    queue_new, label_new = pl.pallas_call(
        body,
        grid=(nblocks,),
        in_specs=[
            pl.BlockSpec((_BW, dim), lambda j: (jnp.minimum(j, nfb - 1), 0)),
            pl.BlockSpec((1, _BW), lambda j: (0, jnp.minimum(j, nfb - 1))),
            pl.BlockSpec((dim, _BW), lambda j: (0, jnp.maximum(j, nfb))),
            pl.BlockSpec((1, _BW), lambda j: (0, jnp.maximum(j, nfb))),
        ],
        out_specs=[
            pl.BlockSpec((dim, _BW), lambda j: (0, j)),
            pl.BlockSpec((1, _BW), lambda j: (0, j)),
        ],
        out_shape=[
            jax.ShapeDtypeStruct((dim, k), queue.dtype),
            jax.ShapeDtypeStruct((1, k), queue_label.dtype),
        ],
        compiler_params=pltpu.CompilerParams(
            dimension_semantics=("parallel",)),
    )(feats, targets2d, queue, queue_label)

    new_ptr = jnp.full((1,), (0 + bsz) % k, dtype=jnp.int32)
    return queue_new, label_new, new_ptr


# speedup vs baseline: 1.0033x; 1.0033x over previous
import functools

import jax
import jax.numpy as jnp
from jax.experimental import pallas as pl
from jax.experimental.pallas import tpu as pltpu

_BR = 128


def _body(feats_ref, tgt_ref, queue_ref, qlab_ref, out_ref, lab_ref, *, bsz):
    i = pl.program_id(0)
    out_ref[:, 0:bsz] = feats_ref[...].T
    out_ref[:, bsz:] = queue_ref[:, bsz:]

    @pl.when(i == 0)
    def _labels():
        lab_ref[:, 0:bsz] = tgt_ref[...]
        lab_ref[:, bsz:] = qlab_ref[:, bsz:]


def kernel(feats, targets, queue, queue_label):
    bsz, dim = feats.shape
    k = queue.shape[1]
    nblocks = dim // _BR
    targets2d = targets.reshape(1, bsz)

    body = functools.partial(_body, bsz=bsz)

    queue_new, label_new = pl.pallas_call(
        body,
        grid=(nblocks,),
        in_specs=[
            pl.BlockSpec((bsz, _BR), lambda i: (0, i)),
            pl.BlockSpec((1, bsz), lambda i: (0, 0)),
            pl.BlockSpec((_BR, k), lambda i: (i, 0)),
            pl.BlockSpec((1, k), lambda i: (0, 0)),
        ],
        out_specs=[
            pl.BlockSpec((_BR, k), lambda i: (i, 0)),
            pl.BlockSpec((1, k), lambda i: (0, 0)),
        ],
        out_shape=[
            jax.ShapeDtypeStruct((dim, k), queue.dtype),
            jax.ShapeDtypeStruct((1, k), queue_label.dtype),
        ],
    )(feats, targets2d, queue, queue_label)

    new_ptr = jnp.full((1,), (0 + bsz) % k, dtype=jnp.int32)
    return queue_new, label_new, new_ptr
